# baseline (speedup 1.0000x reference)
import functools
import math

import jax
import jax.numpy as jnp
import numpy as np
from jax import lax
from jax.experimental import pallas as pl
from jax.experimental.pallas import tpu as pltpu
from jax.experimental.pallas import tpu_sc as plsc

N = 10000
E = 320000
EMB = 128
NB = 2
NG = 50
NF = 128
CUT = 10.0
NC = 119

SC_CORES = 2
SC_SUBCORES = 16
NW = SC_CORES * SC_SUBCORES
EW = E // NW
CH = 80
NCH = EW // CH
RSTEP = 624
RSLAB = 640

TE = 3200
NTE = E // TE
RT = 1000
NRT = N // RT

KPAD = 56
_PERM = np.concatenate([
    np.concatenate([np.arange(0, 32, 2), np.arange(1, 32, 2)]) + 32 * k
    for k in range(4)])

_LOG2 = math.log(2.0)
_DELTA = CUT / (NG - 1)
_COEFF = -0.5 / _DELTA ** 2


def _ssp(v):
    return (jnp.log(1.0 + jnp.exp(jnp.minimum(v, 60.0)))
            + jnp.maximum(v - 60.0, 0.0) - _LOG2)


def _sc_mesh():
    return plsc.VectorSubcoreMesh(
        core_axis_name="c", subcore_axis_name="s",
        num_cores=SC_CORES, num_subcores=SC_SUBCORES)


def _sc_d2_body(px_h, py_h, pz_h, row_h, col_h, d2_h,
                px_v, py_v, pz_v, row_v, col_v, d2_v):
    cid = lax.axis_index("c")
    sid = lax.axis_index("s")
    wid = cid * SC_SUBCORES + sid
    pltpu.sync_copy(px_h, px_v)
    pltpu.sync_copy(py_h, py_v)
    pltpu.sync_copy(pz_h, pz_v)
    base = wid * EW
    pltpu.sync_copy(row_h.at[pl.ds(base, EW)], row_v)
    pltpu.sync_copy(col_h.at[pl.ds(base, EW)], col_v)

    def body(i, carry):
        s = pl.ds(i * 16, 16)
        r = row_v[s]
        c = col_v[s]
        dx = plsc.load_gather(px_v, [r]) - plsc.load_gather(px_v, [c])
        dy = plsc.load_gather(py_v, [r]) - plsc.load_gather(py_v, [c])
        dz = plsc.load_gather(pz_v, [r]) - plsc.load_gather(pz_v, [c])
        d2_v[s] = dx * dx + dy * dy + dz * dz
        return carry

    lax.fori_loop(0, EW // 16, body, 0)
    pltpu.sync_copy(d2_v, d2_h.at[pl.ds(base, EW)])


def _sc_d2(px, py, pz, row, col):
    f = pl.kernel(
        _sc_d2_body,
        out_type=jax.ShapeDtypeStruct((E,), jnp.float32),
        mesh=_sc_mesh(),
        compiler_params=pltpu.CompilerParams(needs_layout_passes=False, use_tc_tiling_on_sc=False),
        scratch_types=[
            pltpu.VMEM((N,), jnp.float32),
            pltpu.VMEM((N,), jnp.float32),
            pltpu.VMEM((N,), jnp.float32),
            pltpu.VMEM((EW,), jnp.int32),
            pltpu.VMEM((EW,), jnp.int32),
            pltpu.VMEM((EW,), jnp.float32),
        ],
    )
    return f(px, py, pz, row, col)


def _sc_msg_body(wc_h, hx_h, row_h, col_h, aggp_h,
                 agg_sh, hxg_v, wc_v, msg_v, row_v, col_v,
                 sem_idx, sem_wc, sem_g):
    cid = lax.axis_index("c")
    sid = lax.axis_index("s")
    wid = cid * SC_SUBCORES + sid
    wbase = wid * EW

    def zbody(r, carry):
        for j in range(EMB // 16):
            msg_v[0, r, pl.ds(j * 16, 16)] = jnp.zeros((16,), jnp.float32)
        return carry

    lax.fori_loop(0, CH, zbody, 0)

    def zcopy(i, carry):
        pltpu.sync_copy(msg_v.at[0], agg_sh.at[pl.ds(sid * RSTEP + i * CH, CH)])
        return carry

    lax.fori_loop(0, RSLAB // CH, zcopy, 0)
    plsc.subcore_barrier()

    def issue_idx(k, p):
        base = wbase + k * CH
        pltpu.async_copy(row_h.at[pl.ds(base, CH)], row_v.at[p], sem_idx.at[p])
        pltpu.async_copy(col_h.at[pl.ds(base, CH)], col_v.at[p], sem_idx.at[p])
        pltpu.async_copy(wc_h.at[pl.ds(base, CH)], wc_v.at[p], sem_wc.at[p])

    def wait_idx(p):
        pltpu.make_async_copy(row_h.at[pl.ds(0, CH)], row_v.at[p],
                              sem_idx.at[p]).wait()
        pltpu.make_async_copy(col_h.at[pl.ds(0, CH)], col_v.at[p],
                              sem_idx.at[p]).wait()

    def issue_gather(p):
        pltpu.async_copy(hx_h.at[row_v.at[p]], hxg_v.at[p], sem_g.at[p])

    def process(p, np_, last):
        if not last:
            wait_idx(np_)
            issue_gather(np_)
        pltpu.make_async_copy(hx_h.at[row_v.at[p]], hxg_v.at[p],
                              sem_g.at[p]).wait()
        pltpu.make_async_copy(wc_h.at[pl.ds(0, CH)], wc_v.at[p],
                              sem_wc.at[p]).wait()

        def mul(r, c2):
            for j in range(EMB // 32):
                s = pl.ds(j * 16, 16)
                xw = wc_v[p, r, s]
                xh = hxg_v[p, r, s]
                wlo = plsc.bitcast(lax.shift_left(xw, 16), jnp.float32)
                whi = plsc.bitcast(jnp.bitwise_and(xw, jnp.int32(-65536)),
                                   jnp.float32)
                hlo = plsc.bitcast(lax.shift_left(xh, 16), jnp.float32)
                hhi = plsc.bitcast(jnp.bitwise_and(xh, jnp.int32(-65536)),
                                   jnp.float32)
                msg_v[p, r, pl.ds(j * 32, 16)] = wlo * hlo
                msg_v[p, r, pl.ds(j * 32 + 16, 16)] = whi * hhi
            return c2

        lax.fori_loop(0, CH, mul, 0)
        pltpu.sync_copy(msg_v.at[p], agg_sh.at[col_v.at[p]], add=True)

    issue_idx(0, 0)
    issue_idx(1, 1)
    wait_idx(0)
    issue_gather(0)

    def pair(g, carry):
        k0 = g * 2
        process(0, 1, False)
        issue_idx(k0 + 2, 0)
        process(1, 0, False)

        @pl.when(k0 + 3 < NCH)
        def _():
            issue_idx(k0 + 3, 1)

        return carry

    lax.fori_loop(0, NCH // 2, pair, 0)
    process(0, 1, True)

    plsc.subcore_barrier()
    pltpu.sync_copy(agg_sh.at[pl.ds(sid * RSTEP, RSLAB)],
                    aggp_h.at[cid, pl.ds(sid * RSTEP, RSLAB)])


def _sc_msg(wc, hx, row, col):
    f = pl.kernel(
        _sc_msg_body,
        out_type=jax.ShapeDtypeStruct((SC_CORES, N, EMB), jnp.float32),
        mesh=_sc_mesh(),
        compiler_params=pltpu.CompilerParams(needs_layout_passes=False, use_tc_tiling_on_sc=False),
        scratch_types=[
            pltpu.VMEM_SHARED((N, EMB), jnp.float32),
            pltpu.VMEM((2, CH, EMB // 2), jnp.int32),
            pltpu.VMEM((2, CH, EMB // 2), jnp.int32),
            pltpu.VMEM((2, CH, EMB), jnp.float32),
            pltpu.VMEM((2, CH), jnp.int32),
            pltpu.VMEM((2, CH), jnp.int32),
            pltpu.SemaphoreType.DMA((2,)),
            pltpu.SemaphoreType.DMA((2,)),
            pltpu.SemaphoreType.DMA((2,)),
        ],
    )
    return f(wc, hx, row, col)


def _tc_wc_body(d2_ref, w1_ref, b1_ref, w2_ref, b2_ref, out_ref):
    d2 = d2_ref[0, 0, :]
    dist = jnp.sqrt(d2 + 1e-12)
    ki = lax.broadcasted_iota(jnp.int32, (1, KPAD), 1)
    offs = jnp.where(ki < NG, ki.astype(jnp.float32) * _DELTA, 0.0)
    dd = dist.reshape(TE, 1) - offs
    ea = jnp.exp(_COEFF * (dd * dd))
    t = jnp.dot(ea, w1_ref[...], preferred_element_type=jnp.float32) + b1_ref[...]
    t = _ssp(t)
    w = jnp.dot(t, w2_ref[...], preferred_element_type=jnp.float32) + b2_ref[...]
    cfac = 0.5 * (jnp.cos(dist * (math.pi / CUT)) + 1.0)
    out_ref[0] = (w * cfac.reshape(TE, 1)).astype(jnp.bfloat16)


def _tc_wc(d2, w1p, b1, w2, b2):
    return pl.pallas_call(
        _tc_wc_body,
        grid=(NTE,),
        in_specs=[
            pl.BlockSpec((1, 1, TE), lambda e: (e, 0, 0)),
            pl.BlockSpec((KPAD, NF), lambda e: (0, 0)),
            pl.BlockSpec((1, NF), lambda e: (0, 0)),
            pl.BlockSpec((NF, NF), lambda e: (0, 0)),
            pl.BlockSpec((1, NF), lambda e: (0, 0)),
        ],
        out_specs=pl.BlockSpec((1, TE, EMB), lambda e: (e, 0, 0)),
        out_shape=jax.ShapeDtypeStruct((NTE, TE, EMB), jnp.bfloat16),
    )(d2, w1p, b1, w2, b2)


def _tc_embed_body(x_ref, emb_ref, l1_ref, h_ref, hx_ref):
    xv = x_ref[0, 0, :]
    oh = (xv.reshape(RT, 1)
          == lax.broadcasted_iota(jnp.int32, (RT, 128), 1)).astype(jnp.float32)
    h = jnp.dot(oh, emb_ref[...], preferred_element_type=jnp.float32)
    h_ref[...] = h
    hx_ref[...] = jnp.dot(
        h, l1_ref[...], preferred_element_type=jnp.float32).astype(jnp.bfloat16)


def _tc_embed(x3, embp, l1):
    return pl.pallas_call(
        _tc_embed_body,
        grid=(NRT,),
        in_specs=[
            pl.BlockSpec((1, 1, RT), lambda i: (i, 0, 0)),
            pl.BlockSpec((128, EMB), lambda i: (0, 0)),
            pl.BlockSpec((EMB, NF), lambda i: (0, 0)),
        ],
        out_specs=[
            pl.BlockSpec((RT, EMB), lambda i: (i, 0)),
            pl.BlockSpec((RT, NF), lambda i: (i, 0)),
        ],
        out_shape=[
            jax.ShapeDtypeStruct((N, EMB), jnp.float32),
            jax.ShapeDtypeStruct((N, NF), jnp.bfloat16),
        ],
    )(x3, embp, l1)


def _tc_nodemlp_body(aggp_ref, w2l_ref, b2l_ref, wl_ref, bl_ref, y_ref, st_ref):
    a = aggp_ref[0] + aggp_ref[1]
    t = _ssp(jnp.dot(a, w2l_ref[...], preferred_element_type=jnp.float32)
             + b2l_ref[...])
    y = jnp.dot(t, wl_ref[...], preferred_element_type=jnp.float32) + bl_ref[...]
    y_ref[...] = y

    @pl.when(pl.program_id(0) == 0)
    def _():
        st_ref[...] = jnp.zeros((8, 128), jnp.float32)

    st_ref[0:1, :] += jnp.sum(y, axis=0, keepdims=True)
    st_ref[1:2, :] += jnp.sum(y * y, axis=0, keepdims=True)


def _tc_nodemlp(aggp, w2l, b2l, wl, bl):
    return pl.pallas_call(
        _tc_nodemlp_body,
        grid=(NRT,),
        in_specs=[
            pl.BlockSpec((SC_CORES, RT, EMB), lambda i: (0, i, 0)),
            pl.BlockSpec((NF, NF), lambda i: (0, 0)),
            pl.BlockSpec((1, NF), lambda i: (0, 0)),
            pl.BlockSpec((NF, EMB), lambda i: (0, 0)),
            pl.BlockSpec((1, EMB), lambda i: (0, 0)),
        ],
        out_specs=[
            pl.BlockSpec((RT, EMB), lambda i: (i, 0)),
            pl.BlockSpec((8, 128), lambda i: (0, 0)),
        ],
        out_shape=[
            jax.ShapeDtypeStruct((N, EMB), jnp.float32),
            jax.ShapeDtypeStruct((8, 128), jnp.float32),
        ],
    )(aggp, w2l, b2l, wl, bl)


def _tc_norm_body(with_hx, y_ref, st_ref, h0_ref, l1_ref, h_ref, hx_ref=None):
    mean = st_ref[0:1, :] * (1.0 / N)
    ex2 = st_ref[1:2, :] * (1.0 / N)
    var = ex2 - mean * mean
    inv = lax.rsqrt(var + 1e-5)
    hn = (y_ref[...] - mean) * inv + h0_ref[...]
    h_ref[...] = hn
    if with_hx:
        hx_ref[...] = jnp.dot(
            hn, l1_ref[...],
            preferred_element_type=jnp.float32).astype(jnp.bfloat16)


def _tc_norm(y, st, h0, l1, with_hx):
    out_specs = [pl.BlockSpec((RT, EMB), lambda i: (i, 0))]
    out_shape = [jax.ShapeDtypeStruct((N, EMB), jnp.float32)]
    if with_hx:
        out_specs.append(pl.BlockSpec((RT, NF), lambda i: (i, 0)))
        out_shape.append(jax.ShapeDtypeStruct((N, NF), jnp.bfloat16))
    res = pl.pallas_call(
        functools.partial(_tc_norm_body, with_hx),
        grid=(NRT,),
        in_specs=[
            pl.BlockSpec((RT, EMB), lambda i: (i, 0)),
            pl.BlockSpec((8, 128), lambda i: (0, 0)),
            pl.BlockSpec((RT, EMB), lambda i: (i, 0)),
            pl.BlockSpec((EMB, NF), lambda i: (0, 0)),
        ],
        out_specs=out_specs,
        out_shape=out_shape,
    )(y, st, h0, l1)
    return res


def kernel(x, positions, batch, edge_index, emb_table, mlp_w1, mlp_b1,
           mlp_w2, mlp_b2, lin1_w, lin2_w, lin2_b, lin_w, lin_b):
    row = edge_index[0]
    col = edge_index[1]
    px = positions[:, 0]
    py = positions[:, 1]
    pz = positions[:, 2]

    d2 = _sc_d2(px, py, pz, row, col)

    d2r = d2.reshape(NTE, 1, TE)
    w1p = jnp.zeros((NB, KPAD, NF), jnp.float32).at[:, :NG, :].set(mlp_w1)

    def _packpairs(a2d, rows):
        return lax.bitcast_convert_type(
            a2d.reshape(rows, EMB // 2, 2), jnp.int32)

    perm = jnp.asarray(_PERM)
    l2p0 = lin2_w[0][perm, :]
    l2p1 = lin2_w[1][perm, :]

    wc0 = _packpairs(_tc_wc(d2r, w1p[0], mlp_b1[0].reshape(1, NF),
                            mlp_w2[0], mlp_b2[0].reshape(1, NF)).reshape(E, EMB), E)

    embp = jnp.zeros((128, EMB), jnp.float32).at[:NC].set(emb_table)
    h0, hx0 = _tc_embed(x.reshape(NRT, 1, RT), embp, lin1_w[0])

    aggp0 = _sc_msg(wc0, _packpairs(hx0, N), row, col)
    wc1 = _packpairs(_tc_wc(d2r, w1p[1], mlp_b1[1].reshape(1, NF),
                            mlp_w2[1], mlp_b2[1].reshape(1, NF)).reshape(E, EMB), E)
    y0, st0 = _tc_nodemlp(aggp0, l2p0, lin2_b[0].reshape(1, NF),
                          lin_w[0], lin_b[0].reshape(1, EMB))
    h1, hx1 = _tc_norm(y0, st0, h0, lin1_w[1], with_hx=True)

    aggp1 = _sc_msg(wc1, _packpairs(hx1, N), row, col)
    y1, st1 = _tc_nodemlp(aggp1, l2p1, lin2_b[1].reshape(1, NF),
                          lin_w[1], lin_b[1].reshape(1, EMB))
    (h2,) = _tc_norm(y1, st1, h0, lin1_w[1], with_hx=False)
    return h2


# speedup vs baseline: 4.0938x; 4.0938x over previous
import functools
import math

import jax
import jax.numpy as jnp
import numpy as np
from jax import lax
from jax.experimental import pallas as pl
from jax.experimental.pallas import tpu as pltpu
from jax.experimental.pallas import tpu_sc as plsc

N = 10000
E = 320000
EMB = 128
NB = 2
NG = 50
NF = 128
CUT = 10.0
NC = 119

SC_CORES = 2
SC_SUBCORES = 16
NW = SC_CORES * SC_SUBCORES
EW = E // NW
CH = 80
NCH = EW // CH
RSTEP = 624
RSLAB = 640

TE = 3200
NTE = E // TE
RT = 1000
NRT = N // RT

KPAD = 56
_PERM = np.concatenate([
    np.concatenate([np.arange(0, 32, 2), np.arange(1, 32, 2)]) + 32 * k
    for k in range(4)])

_LOG2 = math.log(2.0)
_DELTA = CUT / (NG - 1)
_COEFF = -0.5 / _DELTA ** 2


def _ssp(v):
    return (jnp.log(1.0 + jnp.exp(jnp.minimum(v, 60.0)))
            + jnp.maximum(v - 60.0, 0.0) - _LOG2)


def _sc_mesh():
    return plsc.VectorSubcoreMesh(
        core_axis_name="c", subcore_axis_name="s",
        num_cores=SC_CORES, num_subcores=SC_SUBCORES)


def _sc_d2_body(px_h, py_h, pz_h, row_h, col_h, d2_h,
                px_v, py_v, pz_v, row_v, col_v, d2_v):
    cid = lax.axis_index("c")
    sid = lax.axis_index("s")
    wid = cid * SC_SUBCORES + sid
    pltpu.sync_copy(px_h, px_v)
    pltpu.sync_copy(py_h, py_v)
    pltpu.sync_copy(pz_h, pz_v)
    base = wid * EW
    pltpu.sync_copy(row_h.at[pl.ds(base, EW)], row_v)
    pltpu.sync_copy(col_h.at[pl.ds(base, EW)], col_v)

    def body(i, carry):
        s = pl.ds(i * 16, 16)
        r = row_v[s]
        c = col_v[s]
        dx = plsc.load_gather(px_v, [r]) - plsc.load_gather(px_v, [c])
        dy = plsc.load_gather(py_v, [r]) - plsc.load_gather(py_v, [c])
        dz = plsc.load_gather(pz_v, [r]) - plsc.load_gather(pz_v, [c])
        d2_v[s] = dx * dx + dy * dy + dz * dz
        return carry

    lax.fori_loop(0, EW // 16, body, 0)
    pltpu.sync_copy(d2_v, d2_h.at[pl.ds(base, EW)])


def _sc_d2(px, py, pz, row, col):
    f = pl.kernel(
        _sc_d2_body,
        out_type=jax.ShapeDtypeStruct((E,), jnp.float32),
        mesh=_sc_mesh(),
        compiler_params=pltpu.CompilerParams(needs_layout_passes=False, use_tc_tiling_on_sc=False),
        scratch_types=[
            pltpu.VMEM((N,), jnp.float32),
            pltpu.VMEM((N,), jnp.float32),
            pltpu.VMEM((N,), jnp.float32),
            pltpu.VMEM((EW,), jnp.int32),
            pltpu.VMEM((EW,), jnp.int32),
            pltpu.VMEM((EW,), jnp.float32),
        ],
    )
    return f(px, py, pz, row, col)


def _sc_msg_body(wc_h, hx_h, row_h, col_h, aggp_h,
                 agg_sh, hxg_v, wc_v, row_v, col_v, sem_idx, sem_wc, sem_g):
    cid = lax.axis_index("c")
    sid = lax.axis_index("s")
    wid = cid * SC_SUBCORES + sid
    wbase = wid * EW

    def zbody(r, carry):
        for j in range(EMB // 16):
            hxg_v[0, r, pl.ds(j * 16, 16)] = jnp.zeros((16,), jnp.float32)
        return carry

    lax.fori_loop(0, CH, zbody, 0)

    def zcopy(i, carry):
        pltpu.sync_copy(hxg_v.at[0], agg_sh.at[pl.ds(sid * RSTEP + i * CH, CH)])
        return carry

    lax.fori_loop(0, RSLAB // CH, zcopy, 0)
    plsc.subcore_barrier()

    def issue_idx(k, p):
        base = wbase + k * CH
        pltpu.async_copy(row_h.at[pl.ds(base, CH)], row_v.at[p], sem_idx.at[p])
        pltpu.async_copy(col_h.at[pl.ds(base, CH)], col_v.at[p], sem_idx.at[p])
        pltpu.async_copy(wc_h.at[pl.ds(base, CH)], wc_v.at[p], sem_wc.at[p])

    def wait_idx(p):
        pltpu.make_async_copy(row_h.at[pl.ds(0, CH)], row_v.at[p],
                              sem_idx.at[p]).wait()
        pltpu.make_async_copy(col_h.at[pl.ds(0, CH)], col_v.at[p],
                              sem_idx.at[p]).wait()

    def issue_gather(p):
        pltpu.async_copy(hx_h.at[row_v.at[p]], hxg_v.at[p], sem_g.at[p])

    def process(p, np_, last):
        if not last:
            wait_idx(np_)
            issue_gather(np_)
        pltpu.make_async_copy(hx_h.at[row_v.at[p]], hxg_v.at[p],
                              sem_g.at[p]).wait()
        pltpu.make_async_copy(wc_h.at[pl.ds(0, CH)], wc_v.at[p],
                              sem_wc.at[p]).wait()

        def mul(r, c2):
            for j in range(EMB // 16):
                s = pl.ds(j * 16, 16)
                hxg_v[p, r, s] = hxg_v[p, r, s] * wc_v[p, r, s]
            return c2

        lax.fori_loop(0, CH, mul, 0)
        pltpu.sync_copy(hxg_v.at[p], agg_sh.at[col_v.at[p]], add=True)

    issue_idx(0, 0)
    issue_idx(1, 1)
    wait_idx(0)
    issue_gather(0)

    def pair(g, carry):
        k0 = g * 2
        process(0, 1, False)
        issue_idx(k0 + 2, 0)
        process(1, 0, False)

        @pl.when(k0 + 3 < NCH)
        def _():
            issue_idx(k0 + 3, 1)

        return carry

    lax.fori_loop(0, NCH // 2, pair, 0)
    process(0, 1, True)

    plsc.subcore_barrier()
    pltpu.sync_copy(agg_sh.at[pl.ds(sid * RSTEP, RSLAB)],
                    aggp_h.at[cid, pl.ds(sid * RSTEP, RSLAB)])


def _sc_msg(wc, hx, row, col):
    f = pl.kernel(
        _sc_msg_body,
        out_type=jax.ShapeDtypeStruct((SC_CORES, N, EMB), jnp.float32),
        mesh=_sc_mesh(),
        compiler_params=pltpu.CompilerParams(needs_layout_passes=False, use_tc_tiling_on_sc=False),
        scratch_types=[
            pltpu.VMEM_SHARED((N, EMB), jnp.float32),
            pltpu.VMEM((2, CH, EMB), jnp.float32),
            pltpu.VMEM((2, CH, EMB), jnp.float32),
            pltpu.VMEM((2, CH), jnp.int32),
            pltpu.VMEM((2, CH), jnp.int32),
            pltpu.SemaphoreType.DMA((2,)),
            pltpu.SemaphoreType.DMA((2,)),
            pltpu.SemaphoreType.DMA((2,)),
        ],
    )
    return f(wc, hx, row, col)


def _tc_wc_body(d2_ref, w1_ref, b1_ref, w2_ref, b2_ref, out_ref):
    d2 = d2_ref[0, 0, :]
    dist = jnp.sqrt(d2 + 1e-12)
    ki = lax.broadcasted_iota(jnp.int32, (1, KPAD), 1)
    offs = jnp.where(ki < NG, ki.astype(jnp.float32) * _DELTA, 0.0)
    dd = dist.reshape(TE, 1) - offs
    ea = jnp.exp(_COEFF * (dd * dd))
    t = jnp.dot(ea, w1_ref[...], preferred_element_type=jnp.float32) + b1_ref[...]
    t = _ssp(t)
    w = jnp.dot(t, w2_ref[...], preferred_element_type=jnp.float32) + b2_ref[...]
    cfac = 0.5 * (jnp.cos(dist * (math.pi / CUT)) + 1.0)
    out_ref[0] = w * cfac.reshape(TE, 1)


def _tc_wc(d2, w1p, b1, w2, b2):
    return pl.pallas_call(
        _tc_wc_body,
        grid=(NTE,),
        in_specs=[
            pl.BlockSpec((1, 1, TE), lambda e: (e, 0, 0)),
            pl.BlockSpec((KPAD, NF), lambda e: (0, 0)),
            pl.BlockSpec((1, NF), lambda e: (0, 0)),
            pl.BlockSpec((NF, NF), lambda e: (0, 0)),
            pl.BlockSpec((1, NF), lambda e: (0, 0)),
        ],
        out_specs=pl.BlockSpec((1, TE, EMB), lambda e: (e, 0, 0)),
        out_shape=jax.ShapeDtypeStruct((NTE, TE, EMB), jnp.float32),
    )(d2, w1p, b1, w2, b2)


def _tc_embed_body(x_ref, emb_ref, l1_ref, h_ref, hx_ref):
    xv = x_ref[0, 0, :]
    oh = (xv.reshape(RT, 1)
          == lax.broadcasted_iota(jnp.int32, (RT, 128), 1)).astype(jnp.float32)
    h = jnp.dot(oh, emb_ref[...], preferred_element_type=jnp.float32)
    h_ref[...] = h
    hx_ref[...] = jnp.dot(h, l1_ref[...], preferred_element_type=jnp.float32)


def _tc_embed(x3, embp, l1):
    return pl.pallas_call(
        _tc_embed_body,
        grid=(NRT,),
        in_specs=[
            pl.BlockSpec((1, 1, RT), lambda i: (i, 0, 0)),
            pl.BlockSpec((128, EMB), lambda i: (0, 0)),
            pl.BlockSpec((EMB, NF), lambda i: (0, 0)),
        ],
        out_specs=[
            pl.BlockSpec((RT, EMB), lambda i: (i, 0)),
            pl.BlockSpec((RT, NF), lambda i: (i, 0)),
        ],
        out_shape=[
            jax.ShapeDtypeStruct((N, EMB), jnp.float32),
            jax.ShapeDtypeStruct((N, NF), jnp.float32),
        ],
    )(x3, embp, l1)


def _tc_nodemlp_body(aggp_ref, w2l_ref, b2l_ref, wl_ref, bl_ref, y_ref, st_ref):
    a = aggp_ref[0] + aggp_ref[1]
    t = _ssp(jnp.dot(a, w2l_ref[...], preferred_element_type=jnp.float32)
             + b2l_ref[...])
    y = jnp.dot(t, wl_ref[...], preferred_element_type=jnp.float32) + bl_ref[...]
    y_ref[...] = y

    @pl.when(pl.program_id(0) == 0)
    def _():
        st_ref[...] = jnp.zeros((8, 128), jnp.float32)

    st_ref[0:1, :] += jnp.sum(y, axis=0, keepdims=True)
    st_ref[1:2, :] += jnp.sum(y * y, axis=0, keepdims=True)


def _tc_nodemlp(aggp, w2l, b2l, wl, bl):
    return pl.pallas_call(
        _tc_nodemlp_body,
        grid=(NRT,),
        in_specs=[
            pl.BlockSpec((SC_CORES, RT, EMB), lambda i: (0, i, 0)),
            pl.BlockSpec((NF, NF), lambda i: (0, 0)),
            pl.BlockSpec((1, NF), lambda i: (0, 0)),
            pl.BlockSpec((NF, EMB), lambda i: (0, 0)),
            pl.BlockSpec((1, EMB), lambda i: (0, 0)),
        ],
        out_specs=[
            pl.BlockSpec((RT, EMB), lambda i: (i, 0)),
            pl.BlockSpec((8, 128), lambda i: (0, 0)),
        ],
        out_shape=[
            jax.ShapeDtypeStruct((N, EMB), jnp.float32),
            jax.ShapeDtypeStruct((8, 128), jnp.float32),
        ],
    )(aggp, w2l, b2l, wl, bl)


def _tc_norm_body(with_hx, y_ref, st_ref, h0_ref, l1_ref, h_ref, hx_ref=None):
    mean = st_ref[0:1, :] * (1.0 / N)
    ex2 = st_ref[1:2, :] * (1.0 / N)
    var = ex2 - mean * mean
    inv = lax.rsqrt(var + 1e-5)
    hn = (y_ref[...] - mean) * inv + h0_ref[...]
    h_ref[...] = hn
    if with_hx:
        hx_ref[...] = jnp.dot(hn, l1_ref[...], preferred_element_type=jnp.float32)


def _tc_norm(y, st, h0, l1, with_hx):
    out_specs = [pl.BlockSpec((RT, EMB), lambda i: (i, 0))]
    out_shape = [jax.ShapeDtypeStruct((N, EMB), jnp.float32)]
    if with_hx:
        out_specs.append(pl.BlockSpec((RT, NF), lambda i: (i, 0)))
        out_shape.append(jax.ShapeDtypeStruct((N, NF), jnp.float32))
    res = pl.pallas_call(
        functools.partial(_tc_norm_body, with_hx),
        grid=(NRT,),
        in_specs=[
            pl.BlockSpec((RT, EMB), lambda i: (i, 0)),
            pl.BlockSpec((8, 128), lambda i: (0, 0)),
            pl.BlockSpec((RT, EMB), lambda i: (i, 0)),
            pl.BlockSpec((EMB, NF), lambda i: (0, 0)),
        ],
        out_specs=out_specs,
        out_shape=out_shape,
    )(y, st, h0, l1)
    return res


def kernel(x, positions, batch, edge_index, emb_table, mlp_w1, mlp_b1,
           mlp_w2, mlp_b2, lin1_w, lin2_w, lin2_b, lin_w, lin_b):
    row = edge_index[0]
    col = edge_index[1]
    px = positions[:, 0]
    py = positions[:, 1]
    pz = positions[:, 2]

    d2 = _sc_d2(px, py, pz, row, col)

    d2r = d2.reshape(NTE, 1, TE)
    w1p = jnp.zeros((NB, KPAD, NF), jnp.float32).at[:, :NG, :].set(mlp_w1)

    wc0 = _tc_wc(d2r, w1p[0], mlp_b1[0].reshape(1, NF),
                 mlp_w2[0], mlp_b2[0].reshape(1, NF)).reshape(E, EMB)

    embp = jnp.zeros((128, EMB), jnp.float32).at[:NC].set(emb_table)
    h0, hx0 = _tc_embed(x.reshape(NRT, 1, RT), embp, lin1_w[0])

    aggp0 = _sc_msg(wc0, hx0, row, col)
    wc1 = _tc_wc(d2r, w1p[1], mlp_b1[1].reshape(1, NF),
                 mlp_w2[1], mlp_b2[1].reshape(1, NF)).reshape(E, EMB)
    y0, st0 = _tc_nodemlp(aggp0, lin2_w[0], lin2_b[0].reshape(1, NF),
                          lin_w[0], lin_b[0].reshape(1, EMB))
    h1, hx1 = _tc_norm(y0, st0, h0, lin1_w[1], with_hx=True)

    aggp1 = _sc_msg(wc1, hx1, row, col)
    y1, st1 = _tc_nodemlp(aggp1, lin2_w[1], lin2_b[1].reshape(1, NF),
                          lin_w[1], lin_b[1].reshape(1, EMB))
    (h2,) = _tc_norm(y1, st1, h0, lin1_w[1], with_hx=False)
    return h2
